# baseline (speedup 1.0000x reference)
import jax
import jax.numpy as jnp
from jax import lax
from jax.experimental import pallas as pl
from jax.experimental.pallas import tpu as pltpu
from jax.experimental.pallas import tpu_sc as plsc

_INPUT_DIM = 1000000
_NUM_FIELDS = 26
_BATCH = 16384
_NC = 2
_NS = 16
_NW = _NC * _NS
_BPW = _BATCH // _NW
_LANES = 16
_CHUNK = _BPW * _NUM_FIELDS
_WSLICE = 62496
_WTAIL = _INPUT_DIM - _NS * _WSLICE
_Q = _WSLICE // 4


def _sc_body(idx_hbm, w_hbm, bias_hbm, out_hbm, idx_v, vals_v, out_v, b_s,
             w_sp, stg0, stg1, sem_i, sem_g, sem_s0, sem_s1, sem_p):
    cid = lax.axis_index("c")
    sid = lax.axis_index("s")
    wid = sid * _NC + cid
    base = sid * _WSLICE

    cpi = pltpu.async_copy(idx_hbm.at[wid], idx_v, sem_i)
    pltpu.sync_copy(bias_hbm, b_s)

    @pl.when(sid == 0)
    def _():
        pltpu.sync_copy(w_hbm.at[pl.ds(_NS * _WSLICE, _WTAIL)],
                        stg0.at[pl.ds(0, _WTAIL)])
        pltpu.sync_copy(stg0.at[pl.ds(0, _WTAIL)],
                        w_sp.at[pl.ds(_NS * _WSLICE, _WTAIL)])

    h0 = pltpu.async_copy(w_hbm.at[pl.ds(base + 0 * _Q, _Q)], stg0, sem_s0)
    h1 = pltpu.async_copy(w_hbm.at[pl.ds(base + 1 * _Q, _Q)], stg1, sem_s1)
    h0.wait()
    p0 = pltpu.async_copy(stg0, w_sp.at[pl.ds(base + 0 * _Q, _Q)], sem_p)
    h1.wait()
    p1 = pltpu.async_copy(stg1, w_sp.at[pl.ds(base + 1 * _Q, _Q)], sem_p)
    p0.wait()
    h2 = pltpu.async_copy(w_hbm.at[pl.ds(base + 2 * _Q, _Q)], stg0, sem_s0)
    p1.wait()
    h3 = pltpu.async_copy(w_hbm.at[pl.ds(base + 3 * _Q, _Q)], stg1, sem_s1)
    h2.wait()
    p2 = pltpu.async_copy(stg0, w_sp.at[pl.ds(base + 2 * _Q, _Q)], sem_p)
    h3.wait()
    p3 = pltpu.async_copy(stg1, w_sp.at[pl.ds(base + 3 * _Q, _Q)], sem_p)
    p2.wait()
    p3.wait()
    cpi.wait()
    plsc.subcore_barrier()

    _H = _CHUNK // 2
    g0 = pltpu.async_copy(w_sp.at[idx_v.at[pl.ds(0, _H)]],
                          vals_v.at[pl.ds(0, _H)], sem_g)
    g1 = pltpu.async_copy(w_sp.at[idx_v.at[pl.ds(_H, _H)]],
                          vals_v.at[pl.ds(_H, _H)], sem_s0)

    zeros = jnp.zeros((_LANES,), jnp.int32)
    bias_vec = plsc.load_gather(b_s, [zeros])
    nf_half = _NUM_FIELDS // 2

    g0.wait()

    def jbody0(j, carry):
        jbase = j * _LANES
        acc = bias_vec
        for f in range(nf_half):
            acc = acc + vals_v[pl.ds(f * _BPW + jbase, _LANES)]
        out_v[pl.ds(jbase, _LANES)] = acc
        return carry

    lax.fori_loop(0, _BPW // _LANES, jbody0, 0)
    g1.wait()

    def jbody1(j, carry):
        jbase = j * _LANES
        acc = out_v[pl.ds(jbase, _LANES)]
        for f in range(nf_half, _NUM_FIELDS):
            acc = acc + vals_v[pl.ds(f * _BPW + jbase, _LANES)]
        out_v[pl.ds(jbase, _LANES)] = acc
        return carry

    lax.fori_loop(0, _BPW // _LANES, jbody1, 0)
    pltpu.sync_copy(out_v, out_hbm.at[pl.ds(wid * _BPW, _BPW)])


@jax.jit
def _lr_logits(idx, w, b):
    mesh = plsc.VectorSubcoreMesh(core_axis_name="c", subcore_axis_name="s")
    ker = pl.kernel(
        _sc_body,
        out_type=jax.ShapeDtypeStruct((_BATCH,), jnp.float32),
        mesh=mesh,
        scratch_types=[
            pltpu.VMEM((_CHUNK,), jnp.int32),
            pltpu.VMEM((_CHUNK,), jnp.float32),
            pltpu.VMEM((_BPW,), jnp.float32),
            pltpu.VMEM((1,), jnp.float32),
            pltpu.VMEM_SHARED((_INPUT_DIM,), jnp.float32),
            pltpu.VMEM((_Q,), jnp.float32),
            pltpu.VMEM((_Q,), jnp.float32),
            pltpu.SemaphoreType.DMA,
            pltpu.SemaphoreType.DMA,
            pltpu.SemaphoreType.DMA,
            pltpu.SemaphoreType.DMA,
            pltpu.SemaphoreType.DMA,
        ],
        compiler_params=pltpu.CompilerParams(needs_layout_passes=False),
    )
    return ker(idx, w, b)


def kernel(inputs, w, b):
    idx = inputs.reshape(_NW, _BPW, _NUM_FIELDS).transpose(0, 2, 1).reshape(_NW, _CHUNK)
    out = _lr_logits(idx, w, b.astype(jnp.float32))
    return out.reshape(_BATCH, 1)


# speedup vs baseline: 1.0149x; 1.0149x over previous
import jax
import jax.numpy as jnp
from jax import lax
from jax.experimental import pallas as pl
from jax.experimental.pallas import tpu as pltpu
from jax.experimental.pallas import tpu_sc as plsc

_INPUT_DIM = 1000000
_NUM_FIELDS = 26
_BATCH = 16384
_NC = 2
_NS = 16
_NW = _NC * _NS
_BPW = _BATCH // _NW
_LANES = 16
_CHUNK = _BPW * _NUM_FIELDS
_WSLICE = 62496
_WTAIL = _INPUT_DIM - _NS * _WSLICE
_Q = _WSLICE // 4


def _sc_body(idx_hbm, w_hbm, bias_hbm, out_hbm, idx_v, vals_v, out_v, b_s,
             w_sp, stg0, stg1, sem_i, sem_g, sem_s0, sem_s1, sem_p):
    cid = lax.axis_index("c")
    sid = lax.axis_index("s")
    wid = sid * _NC + cid
    base = sid * _WSLICE

    h0 = pltpu.async_copy(w_hbm.at[pl.ds(base + 0 * _Q, _Q)], stg0, sem_s0)
    h1 = pltpu.async_copy(w_hbm.at[pl.ds(base + 1 * _Q, _Q)], stg1, sem_s1)
    cpi = pltpu.async_copy(idx_hbm.at[wid], idx_v, sem_i)
    pltpu.sync_copy(bias_hbm, b_s)
    h0.wait()
    p0 = pltpu.async_copy(stg0, w_sp.at[pl.ds(base + 0 * _Q, _Q)], sem_p)
    h1.wait()
    p1 = pltpu.async_copy(stg1, w_sp.at[pl.ds(base + 1 * _Q, _Q)], sem_p)
    p0.wait()
    h2 = pltpu.async_copy(w_hbm.at[pl.ds(base + 2 * _Q, _Q)], stg0, sem_s0)
    p1.wait()
    h3 = pltpu.async_copy(w_hbm.at[pl.ds(base + 3 * _Q, _Q)], stg1, sem_s1)
    h2.wait()
    p2 = pltpu.async_copy(stg0, w_sp.at[pl.ds(base + 2 * _Q, _Q)], sem_p)
    h3.wait()
    p3 = pltpu.async_copy(stg1, w_sp.at[pl.ds(base + 3 * _Q, _Q)], sem_p)
    p2.wait()
    p3.wait()

    @pl.when(sid == 0)
    def _():
        pltpu.sync_copy(w_hbm.at[pl.ds(_NS * _WSLICE, _WTAIL)],
                        stg0.at[pl.ds(0, _WTAIL)])
        pltpu.sync_copy(stg0.at[pl.ds(0, _WTAIL)],
                        w_sp.at[pl.ds(_NS * _WSLICE, _WTAIL)])

    cpi.wait()
    plsc.subcore_barrier()

    _H = _CHUNK // 2
    g0 = pltpu.async_copy(w_sp.at[idx_v.at[pl.ds(0, _H)]],
                          vals_v.at[pl.ds(0, _H)], sem_g)
    g1 = pltpu.async_copy(w_sp.at[idx_v.at[pl.ds(_H, _H)]],
                          vals_v.at[pl.ds(_H, _H)], sem_s0)

    zeros = jnp.zeros((_LANES,), jnp.int32)
    bias_vec = plsc.load_gather(b_s, [zeros])
    nf_half = _NUM_FIELDS // 2

    g0.wait()

    def jbody0(j, carry):
        jbase = j * _LANES
        acc = bias_vec
        for f in range(nf_half):
            acc = acc + vals_v[pl.ds(f * _BPW + jbase, _LANES)]
        out_v[pl.ds(jbase, _LANES)] = acc
        return carry

    lax.fori_loop(0, _BPW // _LANES, jbody0, 0)
    g1.wait()

    def jbody1(j, carry):
        jbase = j * _LANES
        acc = out_v[pl.ds(jbase, _LANES)]
        for f in range(nf_half, _NUM_FIELDS):
            acc = acc + vals_v[pl.ds(f * _BPW + jbase, _LANES)]
        out_v[pl.ds(jbase, _LANES)] = acc
        return carry

    lax.fori_loop(0, _BPW // _LANES, jbody1, 0)
    pltpu.sync_copy(out_v, out_hbm.at[pl.ds(wid * _BPW, _BPW)])


@jax.jit
def _lr_logits(idx, w, b):
    mesh = plsc.VectorSubcoreMesh(core_axis_name="c", subcore_axis_name="s")
    ker = pl.kernel(
        _sc_body,
        out_type=jax.ShapeDtypeStruct((_BATCH,), jnp.float32),
        mesh=mesh,
        scratch_types=[
            pltpu.VMEM((_CHUNK,), jnp.int32),
            pltpu.VMEM((_CHUNK,), jnp.float32),
            pltpu.VMEM((_BPW,), jnp.float32),
            pltpu.VMEM((1,), jnp.float32),
            pltpu.VMEM_SHARED((_INPUT_DIM,), jnp.float32),
            pltpu.VMEM((_Q,), jnp.float32),
            pltpu.VMEM((_Q,), jnp.float32),
            pltpu.SemaphoreType.DMA,
            pltpu.SemaphoreType.DMA,
            pltpu.SemaphoreType.DMA,
            pltpu.SemaphoreType.DMA,
            pltpu.SemaphoreType.DMA,
        ],
        compiler_params=pltpu.CompilerParams(needs_layout_passes=False),
    )
    return ker(idx, w, b)


def kernel(inputs, w, b):
    idx = inputs.reshape(_NW, _BPW, _NUM_FIELDS).transpose(0, 2, 1).reshape(_NW, _CHUNK)
    out = _lr_logits(idx, w, b.astype(jnp.float32))
    return out.reshape(_BATCH, 1)


# speedup vs baseline: 1.0295x; 1.0144x over previous
import jax
import jax.numpy as jnp
from jax import lax
from jax.experimental import pallas as pl
from jax.experimental.pallas import tpu as pltpu
from jax.experimental.pallas import tpu_sc as plsc

_INPUT_DIM = 1000000
_NUM_FIELDS = 26
_BATCH = 16384
_NC = 2
_NS = 16
_NW = _NC * _NS
_BPW = _BATCH // _NW
_LANES = 16
_CHUNK = _BPW * _NUM_FIELDS
_WSLICE = 62496
_WTAIL = _INPUT_DIM - _NS * _WSLICE
_Q = _WSLICE // 4


def _sc_body(idx_hbm, w_hbm, bias_hbm, out_hbm, idx_v, vals_v, out_v, b_s,
             w_sp, stg0, stg1, sem_i, sem_g, sem_s0, sem_s1, sem_p):
    cid = lax.axis_index("c")
    sid = lax.axis_index("s")
    wid = sid * _NC + cid
    base = sid * _WSLICE

    h0 = pltpu.async_copy(w_hbm.at[pl.ds(base + 0 * _Q, _Q)], stg0, sem_s0)
    h1 = pltpu.async_copy(w_hbm.at[pl.ds(base + 1 * _Q, _Q)], stg1, sem_s1)
    cpi = pltpu.async_copy(idx_hbm.at[wid], idx_v, sem_i)
    pltpu.sync_copy(bias_hbm, b_s)
    h0.wait()
    p0 = pltpu.async_copy(stg0, w_sp.at[pl.ds(base + 0 * _Q, _Q)], sem_p)
    h1.wait()
    p1 = pltpu.async_copy(stg1, w_sp.at[pl.ds(base + 1 * _Q, _Q)], sem_p)
    p0.wait()
    h2 = pltpu.async_copy(w_hbm.at[pl.ds(base + 2 * _Q, _Q)], stg0, sem_s0)
    p1.wait()
    h3 = pltpu.async_copy(w_hbm.at[pl.ds(base + 3 * _Q, _Q)], stg1, sem_s1)
    h2.wait()
    p2 = pltpu.async_copy(stg0, w_sp.at[pl.ds(base + 2 * _Q, _Q)], sem_p)
    h3.wait()
    p3 = pltpu.async_copy(stg1, w_sp.at[pl.ds(base + 3 * _Q, _Q)], sem_p)
    p2.wait()
    p3.wait()

    @pl.when(sid == 0)
    def _():
        pltpu.sync_copy(w_hbm.at[pl.ds(_NS * _WSLICE, _WTAIL)],
                        stg0.at[pl.ds(0, _WTAIL)])
        pltpu.sync_copy(stg0.at[pl.ds(0, _WTAIL)],
                        w_sp.at[pl.ds(_NS * _WSLICE, _WTAIL)])

    cpi.wait()
    plsc.subcore_barrier()

    fsplits = (0, 7, 13, 20, _NUM_FIELDS)
    gsems = (sem_g, sem_s0, sem_s1, sem_p)
    gs = []
    for k in range(4):
        lo, hi = fsplits[k] * _BPW, fsplits[k + 1] * _BPW
        gs.append(pltpu.async_copy(w_sp.at[idx_v.at[pl.ds(lo, hi - lo)]],
                                   vals_v.at[pl.ds(lo, hi - lo)], gsems[k]))

    zeros = jnp.zeros((_LANES,), jnp.int32)
    bias_vec = plsc.load_gather(b_s, [zeros])

    for k in range(4):
        gs[k].wait()
        f_lo, f_hi = fsplits[k], fsplits[k + 1]

        def jbody(j, carry, f_lo=f_lo, f_hi=f_hi, first=(k == 0)):
            jbase = j * _LANES
            acc = bias_vec if first else out_v[pl.ds(jbase, _LANES)]
            for f in range(f_lo, f_hi):
                acc = acc + vals_v[pl.ds(f * _BPW + jbase, _LANES)]
            out_v[pl.ds(jbase, _LANES)] = acc
            return carry

        lax.fori_loop(0, _BPW // _LANES, jbody, 0)

    pltpu.sync_copy(out_v, out_hbm.at[pl.ds(wid * _BPW, _BPW)])


@jax.jit
def _lr_logits(idx, w, b):
    mesh = plsc.VectorSubcoreMesh(core_axis_name="c", subcore_axis_name="s")
    ker = pl.kernel(
        _sc_body,
        out_type=jax.ShapeDtypeStruct((_BATCH,), jnp.float32),
        mesh=mesh,
        scratch_types=[
            pltpu.VMEM((_CHUNK,), jnp.int32),
            pltpu.VMEM((_CHUNK,), jnp.float32),
            pltpu.VMEM((_BPW,), jnp.float32),
            pltpu.VMEM((1,), jnp.float32),
            pltpu.VMEM_SHARED((_INPUT_DIM,), jnp.float32),
            pltpu.VMEM((_Q,), jnp.float32),
            pltpu.VMEM((_Q,), jnp.float32),
            pltpu.SemaphoreType.DMA,
            pltpu.SemaphoreType.DMA,
            pltpu.SemaphoreType.DMA,
            pltpu.SemaphoreType.DMA,
            pltpu.SemaphoreType.DMA,
        ],
        compiler_params=pltpu.CompilerParams(needs_layout_passes=False),
    )
    return ker(idx, w, b)


def kernel(inputs, w, b):
    idx = inputs.reshape(_NW, _BPW, _NUM_FIELDS).transpose(0, 2, 1).reshape(_NW, _CHUNK)
    out = _lr_logits(idx, w, b.astype(jnp.float32))
    return out.reshape(_BATCH, 1)
